# Initial kernel scaffold; baseline (speedup 1.0000x reference)
import jax
import jax.numpy as jnp
from jax import lax
from jax.experimental import pallas as pl
from jax.experimental.pallas import tpu as pltpu
from jax.experimental.pallas import tpu_sc as plsc

N_NODES = 10000
N_EDGES = 320000
D_FEAT = 128

NC = 2
NS = 16
NW = NC * NS
EW = N_EDGES // NW
B = 80
NCHUNK = EW // B
RPT = N_NODES // NS
ZR = 125


def _sc_body(x_hbm, sidx_hbm, tidx_hbm, en_hbm, es_hbm, part_hbm,
             sidx_v, tidx_v, en_v, es_v, w_v, rows_v, zero_v, acc_ref, sem):
    cid = lax.axis_index("c")
    sid = lax.axis_index("s")
    wid = cid * NS + sid

    def _zrow(i, _):
        for g in range(D_FEAT // 16):
            zero_v[i, pl.ds(g * 16, 16)] = jnp.zeros((16,), jnp.float32)
        return 0
    lax.fori_loop(0, ZR, _zrow, 0)

    for k in range(RPT // ZR):
        pltpu.sync_copy(zero_v, acc_ref.at[pl.ds(sid * RPT + k * ZR, ZR)])

    plsc.subcore_barrier()

    ebase = wid * EW

    def _chunk(c, _):
        base = ebase + c * B
        pltpu.sync_copy(sidx_hbm.at[pl.ds(base, B)], sidx_v)
        pltpu.sync_copy(tidx_hbm.at[pl.ds(base, B)], tidx_v)
        pltpu.sync_copy(en_hbm.at[pl.ds(base, B)], en_v)
        pltpu.sync_copy(es_hbm.at[pl.ds(base, B)], es_v)
        for v in range(B // 16):
            sl = pl.ds(v * 16, 16)
            w_v[sl] = en_v[sl] * es_v[sl]
        pltpu.async_copy(x_hbm.at[sidx_v], rows_v, sem).wait()

        def _scale(j, _):
            w = w_v[j]
            for g in range(D_FEAT // 16):
                sl = pl.ds(g * 16, 16)
                rows_v[j, sl] = rows_v[j, sl] * w
            return 0
        lax.fori_loop(0, B, _scale, 0)

        pltpu.sync_copy(rows_v, acc_ref.at[tidx_v], add=True)
        return 0

    lax.fori_loop(0, NCHUNK, _chunk, 0)

    plsc.subcore_barrier()

    pltpu.sync_copy(acc_ref.at[pl.ds(sid * RPT, RPT)],
                    part_hbm.at[cid, pl.ds(sid * RPT, RPT)])


def _make_sc_kernel():
    mesh = plsc.VectorSubcoreMesh(core_axis_name="c", subcore_axis_name="s")
    return pl.kernel(
        _sc_body,
        out_type=jax.ShapeDtypeStruct((NC, N_NODES, D_FEAT), jnp.float32),
        mesh=mesh,
        scratch_types=[
            pltpu.VMEM((B,), jnp.int32),
            pltpu.VMEM((B,), jnp.int32),
            pltpu.VMEM((B,), jnp.float32),
            pltpu.VMEM((B,), jnp.float32),
            pltpu.VMEM((B,), jnp.float32),
            pltpu.VMEM((B, D_FEAT), jnp.float32),
            pltpu.VMEM((ZR, D_FEAT), jnp.float32),
            pltpu.VMEM_SHARED((N_NODES, D_FEAT), jnp.float32),
            pltpu.SemaphoreType.DMA,
        ],
    )


def _sum2_body(p_ref, o_ref):
    o_ref[...] = p_ref[0] + p_ref[1]


def _tc_sum(partial):
    rows_blk = 1000
    return pl.pallas_call(
        _sum2_body,
        grid=(N_NODES // rows_blk,),
        in_specs=[pl.BlockSpec((NC, rows_blk, D_FEAT), lambda i: (0, i, 0))],
        out_specs=pl.BlockSpec((rows_blk, D_FEAT), lambda i: (i, 0)),
        out_shape=jax.ShapeDtypeStruct((N_NODES, D_FEAT), jnp.float32),
    )(partial)


@jax.jit
def kernel(inputs, eidx, enorm, esgn):
    sidx = eidx[0].astype(jnp.int32)
    tidx = eidx[1].astype(jnp.int32)
    partial = _make_sc_kernel()(inputs, sidx, tidx, enorm, esgn)
    return _tc_sum(partial)


# speedup vs baseline: 4.0434x; 4.0434x over previous
import jax
import jax.numpy as jnp
from jax import lax
from jax.experimental import pallas as pl
from jax.experimental.pallas import tpu as pltpu
from jax.experimental.pallas import tpu_sc as plsc

N_NODES = 10000
N_EDGES = 320000
D_FEAT = 128

NC = 2
NS = 16
NW = NC * NS
EW = N_EDGES // NW
B = 80
NCHUNK = EW // B
STRIPE = 624
TAIL = N_NODES - NS * STRIPE
ZR = 104


def _sc_body(x_hbm, sidx_hbm, tidx_hbm, en_hbm, es_hbm, part_hbm,
             sidx_v, tidx_v, en_v, es_v, w_v, rows_v, zero_v, acc_ref, sem):
    cid = lax.axis_index("c")
    sid = lax.axis_index("s")
    wid = cid * NS + sid

    def _zrow(i, _):
        for g in range(D_FEAT // 16):
            zero_v[i, pl.ds(g * 16, 16)] = jnp.zeros((16,), jnp.float32)
        return 0
    lax.fori_loop(0, ZR, _zrow, 0)

    for k in range(STRIPE // ZR):
        pltpu.sync_copy(zero_v, acc_ref.at[pl.ds(sid * STRIPE + k * ZR, ZR)])

    @pl.when(sid == NS - 1)
    def _zero_tail():
        pltpu.sync_copy(zero_v.at[pl.ds(0, TAIL)],
                        acc_ref.at[pl.ds(NS * STRIPE, TAIL)])

    plsc.subcore_barrier()

    ebase = wid * EW

    def _chunk(c, _):
        base = ebase + c * B
        pltpu.sync_copy(sidx_hbm.at[pl.ds(base, B)], sidx_v)
        pltpu.sync_copy(tidx_hbm.at[pl.ds(base, B)], tidx_v)
        pltpu.sync_copy(en_hbm.at[pl.ds(base, B)], en_v)
        pltpu.sync_copy(es_hbm.at[pl.ds(base, B)], es_v)
        for v in range(B // 16):
            sl = pl.ds(v * 16, 16)
            w_v[sl] = en_v[sl] * es_v[sl]
        pltpu.async_copy(x_hbm.at[sidx_v], rows_v, sem).wait()

        def _scale(v, _):
            w16 = w_v[pl.ds(v * 16, 16)]
            for j in range(16):
                w = w16[j]
                row = v * 16 + j
                for g in range(D_FEAT // 16):
                    sl = pl.ds(g * 16, 16)
                    rows_v[row, sl] = rows_v[row, sl] * w
            return 0
        lax.fori_loop(0, B // 16, _scale, 0)

        pltpu.sync_copy(rows_v, acc_ref.at[tidx_v], add=True)
        return 0

    lax.fori_loop(0, NCHUNK, _chunk, 0)

    plsc.subcore_barrier()

    pltpu.sync_copy(acc_ref.at[pl.ds(sid * STRIPE, STRIPE)],
                    part_hbm.at[cid, pl.ds(sid * STRIPE, STRIPE)])

    @pl.when(sid == NS - 1)
    def _write_tail():
        pltpu.sync_copy(acc_ref.at[pl.ds(NS * STRIPE, TAIL)],
                        part_hbm.at[cid, pl.ds(NS * STRIPE, TAIL)])


def _make_sc_kernel():
    mesh = plsc.VectorSubcoreMesh(core_axis_name="c", subcore_axis_name="s")
    return pl.kernel(
        _sc_body,
        out_type=jax.ShapeDtypeStruct((NC, N_NODES, D_FEAT), jnp.float32),
        mesh=mesh,
        scratch_types=[
            pltpu.VMEM((B,), jnp.int32),
            pltpu.VMEM((B,), jnp.int32),
            pltpu.VMEM((B,), jnp.float32),
            pltpu.VMEM((B,), jnp.float32),
            pltpu.VMEM((B,), jnp.float32),
            pltpu.VMEM((B, D_FEAT), jnp.float32),
            pltpu.VMEM((ZR, D_FEAT), jnp.float32),
            pltpu.VMEM_SHARED((N_NODES, D_FEAT), jnp.float32),
            pltpu.SemaphoreType.DMA,
        ],
    )


def _sum2_body(p_ref, o_ref):
    o_ref[...] = p_ref[0] + p_ref[1]


def _tc_sum(partial):
    rows_blk = 1000
    return pl.pallas_call(
        _sum2_body,
        grid=(N_NODES // rows_blk,),
        in_specs=[pl.BlockSpec((NC, rows_blk, D_FEAT), lambda i: (0, i, 0))],
        out_specs=pl.BlockSpec((rows_blk, D_FEAT), lambda i: (i, 0)),
        out_shape=jax.ShapeDtypeStruct((N_NODES, D_FEAT), jnp.float32),
    )(partial)


@jax.jit
def kernel(inputs, eidx, enorm, esgn):
    sidx = eidx[0].astype(jnp.int32)
    tidx = eidx[1].astype(jnp.int32)
    partial = _make_sc_kernel()(inputs, sidx, tidx, enorm, esgn)
    return _tc_sum(partial)


# speedup vs baseline: 11.2462x; 2.7814x over previous
import jax
import jax.numpy as jnp
from jax import lax
from jax.experimental import pallas as pl
from jax.experimental.pallas import tpu as pltpu
from jax.experimental.pallas import tpu_sc as plsc

N_NODES = 10000
N_EDGES = 320000
D_FEAT = 128

NC = 2
NS = 16
NW = NC * NS
EW = N_EDGES // NW
B = 80
NCHUNK = EW // B
STRIPE = 624
TAIL = N_NODES - NS * STRIPE


def _sc_body(x_hbm, sidx_hbm, tidx_hbm, en_hbm, es_hbm, part_hbm,
             sidx_v, en_v, es_v, rows0_v, rows1_v, tidx0_v, tidx1_v,
             acc_ref, gsem0, gsem1, tsem0, tsem1):
    cid = lax.axis_index("c")
    sid = lax.axis_index("s")
    wid = cid * NS + sid
    ebase = wid * EW
    rows = (rows0_v, rows1_v)
    tidx = (tidx0_v, tidx1_v)
    gsem = (gsem0, gsem1)
    tsem = (tsem0, tsem1)

    pltpu.sync_copy(sidx_hbm.at[pl.ds(ebase, EW)], sidx_v)
    pltpu.sync_copy(en_hbm.at[pl.ds(ebase, EW)], en_v)
    pltpu.sync_copy(es_hbm.at[pl.ds(ebase, EW)], es_v)

    def _zrow(i, _):
        for g in range(D_FEAT // 16):
            rows0_v[i, pl.ds(g * 16, 16)] = jnp.zeros((16,), jnp.float32)
        return 0
    lax.fori_loop(0, B, _zrow, 0)

    for k in range(7):
        pltpu.sync_copy(rows0_v, acc_ref.at[pl.ds(sid * STRIPE + k * B, B)])
    pltpu.sync_copy(rows0_v.at[pl.ds(0, 64)],
                    acc_ref.at[pl.ds(sid * STRIPE + 7 * B, 64)])

    @pl.when(sid == NS - 1)
    def _zero_tail():
        pltpu.sync_copy(rows0_v.at[pl.ds(0, TAIL)],
                        acc_ref.at[pl.ds(NS * STRIPE, TAIL)])

    plsc.subcore_barrier()

    def _issue(c, buf):
        pltpu.async_copy(tidx_hbm.at[pl.ds(ebase + c * B, B)],
                         tidx[buf], tsem[buf])
        pltpu.async_copy(x_hbm.at[sidx_v.at[pl.ds(c * B, B)]],
                         rows[buf], gsem[buf])

    def _process(c, buf):
        pltpu.make_async_copy(x_hbm.at[sidx_v.at[pl.ds(c * B, B)]],
                              rows[buf], gsem[buf]).wait()

        def _scale(v, _):
            sl = pl.ds(c * B + v * 16, 16)
            w16 = en_v[sl] * es_v[sl]
            for j in range(16):
                w = w16[j]
                row = v * 16 + j
                for g in range(D_FEAT // 16):
                    slg = pl.ds(g * 16, 16)
                    rows[buf][row, slg] = rows[buf][row, slg] * w
            return 0
        lax.fori_loop(0, B // 16, _scale, 0)

        pltpu.make_async_copy(tidx_hbm.at[pl.ds(ebase + c * B, B)],
                              tidx[buf], tsem[buf]).wait()
        pltpu.sync_copy(rows[buf], acc_ref.at[tidx[buf]], add=True)

    _issue(0, 0)

    def _step(c2, _):
        c = c2 * 2
        _issue(c + 1, 1)
        _process(c, 0)
        _issue(c + 2, 0)
        _process(c + 1, 1)
        return 0
    lax.fori_loop(0, (NCHUNK - 1) // 2, _step, 0)

    _process(NCHUNK - 1, 0)

    plsc.subcore_barrier()

    pltpu.sync_copy(acc_ref.at[pl.ds(sid * STRIPE, STRIPE)],
                    part_hbm.at[cid, pl.ds(sid * STRIPE, STRIPE)])

    @pl.when(sid == NS - 1)
    def _write_tail():
        pltpu.sync_copy(acc_ref.at[pl.ds(NS * STRIPE, TAIL)],
                        part_hbm.at[cid, pl.ds(NS * STRIPE, TAIL)])


def _make_sc_kernel():
    mesh = plsc.VectorSubcoreMesh(core_axis_name="c", subcore_axis_name="s")
    return pl.kernel(
        _sc_body,
        out_type=jax.ShapeDtypeStruct((NC, N_NODES, D_FEAT), jnp.float32),
        mesh=mesh,
        scratch_types=[
            pltpu.VMEM((EW,), jnp.int32),
            pltpu.VMEM((EW,), jnp.float32),
            pltpu.VMEM((EW,), jnp.float32),
            pltpu.VMEM((B, D_FEAT), jnp.float32),
            pltpu.VMEM((B, D_FEAT), jnp.float32),
            pltpu.VMEM((B,), jnp.int32),
            pltpu.VMEM((B,), jnp.int32),
            pltpu.VMEM_SHARED((N_NODES, D_FEAT), jnp.float32),
            pltpu.SemaphoreType.DMA,
            pltpu.SemaphoreType.DMA,
            pltpu.SemaphoreType.DMA,
            pltpu.SemaphoreType.DMA,
        ],
    )


def _sum2_body(p_ref, o_ref):
    o_ref[...] = p_ref[0] + p_ref[1]


def _tc_sum(partial):
    rows_blk = 1000
    return pl.pallas_call(
        _sum2_body,
        grid=(N_NODES // rows_blk,),
        in_specs=[pl.BlockSpec((NC, rows_blk, D_FEAT), lambda i: (0, i, 0))],
        out_specs=pl.BlockSpec((rows_blk, D_FEAT), lambda i: (i, 0)),
        out_shape=jax.ShapeDtypeStruct((N_NODES, D_FEAT), jnp.float32),
    )(partial)


@jax.jit
def kernel(inputs, eidx, enorm, esgn):
    sidx = eidx[0].astype(jnp.int32)
    tidx = eidx[1].astype(jnp.int32)
    partial = _make_sc_kernel()(inputs, sidx, tidx, enorm, esgn)
    return _tc_sum(partial)
